# Initial kernel scaffold; baseline (speedup 1.0000x reference)
import jax
import jax.numpy as jnp
from jax.experimental import pallas as pl


def kernel(x, W_router, norm_w, wd, wu, gate_w, up_w, down_w):
    raise NotImplementedError("write your pallas kernel here")


# speedup vs baseline: 1.5691x; 1.5691x over previous
import functools

import jax
import jax.numpy as jnp
from jax.experimental import pallas as pl
from jax.experimental.pallas import tpu as pltpu

D = 768
E = 8
N = 2048
DFFN = int(D * 1.618)
DFFN_P = 1280
THRESHOLD = 0.8
MAX_K = 4
EPS = 1e-6

BN_R = 256
BN_D = 512


def _silu(x):
    return x * jax.nn.sigmoid(x)


def _router_body(x_ref, wr_ref, gates_ref, xn_ref):
    x = x_ref[...]
    ms = jnp.mean(x * x, axis=1, keepdims=True)
    xn_ref[...] = x * jax.lax.rsqrt(ms + EPS)
    logits = jnp.dot(x, wr_ref[...], preferred_element_type=jnp.float32)
    m = jnp.max(logits, axis=1, keepdims=True)
    ex = jnp.exp(logits - m)
    probs = ex / jnp.sum(ex, axis=1, keepdims=True)
    cols = []
    for e in range(E):
        pe = probs[:, e:e + 1]
        rank = jnp.zeros_like(pe, dtype=jnp.int32)
        cb = jnp.zeros_like(pe)
        for j in range(E):
            if j == e:
                continue
            pj = probs[:, j:j + 1]
            if j < e:
                beats = pj >= pe
            else:
                beats = pj > pe
            rank = rank + beats.astype(jnp.int32)
            cb = cb + jnp.where(beats, pj, 0.0)
        sel = ((cb < THRESHOLD) & (rank < MAX_K)) | (rank == 0)
        cols.append(jnp.where(sel, pe, 0.0))
    gates_ref[...] = jnp.concatenate(cols, axis=1)


def _router(x, W_router):
    return pl.pallas_call(
        _router_body,
        grid=(N // BN_R,),
        in_specs=[
            pl.BlockSpec((BN_R, D), lambda n: (n, 0)),
            pl.BlockSpec((D, E), lambda n: (0, 0)),
        ],
        out_specs=[
            pl.BlockSpec((BN_R, E), lambda n: (n, 0)),
            pl.BlockSpec((BN_R, D), lambda n: (n, 0)),
        ],
        out_shape=[
            jax.ShapeDtypeStruct((N, E), jnp.float32),
            jax.ShapeDtypeStruct((N, D), jnp.float32),
        ],
    )(x, W_router)


def _dense_body(gates_ref, xn_ref, nw_ref, wd_ref, wu_ref, gw_ref, uw_ref,
                dw_ref, out_ref, acc_ref):
    e = pl.program_id(0)
    n = pl.program_id(1)
    h = xn_ref[...] * nw_ref[0]
    a = _silu(jnp.dot(h, wd_ref[0], preferred_element_type=jnp.float32))
    g = jax.nn.sigmoid(jnp.dot(a, wu_ref[0], preferred_element_type=jnp.float32))
    u = _silu(jnp.dot(h, gw_ref[0], preferred_element_type=jnp.float32)) * \
        jnp.dot(h, uw_ref[0], preferred_element_type=jnp.float32)
    y = jnp.dot(u, dw_ref[0], preferred_element_type=jnp.float32)
    lane = jax.lax.broadcasted_iota(jnp.int32, (BN_D, E), 1)
    gcol = jnp.sum(jnp.where(lane == e, gates_ref[...], 0.0), axis=1,
                   keepdims=True)
    contrib = gcol * (g * y)

    @pl.when(e == 0)
    def _():
        acc_ref[pl.ds(n * BN_D, BN_D), :] = contrib

    @pl.when(e > 0)
    def _():
        acc_ref[pl.ds(n * BN_D, BN_D), :] = (
            acc_ref[pl.ds(n * BN_D, BN_D), :] + contrib)

    @pl.when(e == E - 1)
    def _():
        out_ref[...] = acc_ref[pl.ds(n * BN_D, BN_D), :]


def _dense_experts(gates, xn, norm_w, wd, wu, gw_p, uw_p, dw_p):
    return pl.pallas_call(
        _dense_body,
        grid=(E, N // BN_D),
        in_specs=[
            pl.BlockSpec((BN_D, E), lambda e, n: (n, 0)),
            pl.BlockSpec((BN_D, D), lambda e, n: (n, 0)),
            pl.BlockSpec((1, 1, D), lambda e, n: (e, 0, 0)),
            pl.BlockSpec((1, D, D), lambda e, n: (e, 0, 0)),
            pl.BlockSpec((1, D, D), lambda e, n: (e, 0, 0)),
            pl.BlockSpec((1, D, DFFN_P), lambda e, n: (e, 0, 0)),
            pl.BlockSpec((1, D, DFFN_P), lambda e, n: (e, 0, 0)),
            pl.BlockSpec((1, DFFN_P, D), lambda e, n: (e, 0, 0)),
        ],
        out_specs=pl.BlockSpec((BN_D, D), lambda e, n: (n, 0)),
        out_shape=jax.ShapeDtypeStruct((N, D), jnp.float32),
        scratch_shapes=[pltpu.VMEM((N, D), jnp.float32)],
    )(gates, xn, norm_w, wd, wu, gw_p, uw_p, dw_p)


def kernel(x, W_router, norm_w, wd, wu, gate_w, up_w, down_w):
    gates, xn = _router(x, W_router)
    norm_w = norm_w.reshape(E, 1, D)
    pad = DFFN_P - DFFN
    gw_p = jnp.pad(gate_w, ((0, 0), (0, 0), (0, pad)))
    uw_p = jnp.pad(up_w, ((0, 0), (0, 0), (0, pad)))
    dw_p = jnp.pad(down_w, ((0, 0), (0, pad), (0, 0)))
    return _dense_experts(gates, xn, norm_w, wd, wu, gw_p, uw_p, dw_p)
